# baseline (speedup 1.0000x reference)
import functools

import jax
import jax.numpy as jnp
from jax import lax
from jax.experimental import pallas as pl
from jax.experimental.pallas import tpu as pltpu
from jax.experimental.pallas import tpu_sc as plsc

D_MODEL = 64
SCALE = 8.0

NC = 2
NS = 16
NW = NC * NS

CHUNK = 1024


def _tc_unpack_scale_out(out_pairs, S, B):
    half = B // 2

    def body(in_ref, out_ref):
        x = in_ref[...]
        lo = x[:, :D_MODEL].T
        hi = x[:, D_MODEL:].T
        out_ref[...] = (jnp.concatenate([lo, hi], axis=1) * SCALE).reshape(
            1, D_MODEL, B
        )

    return pl.pallas_call(
        body,
        grid=(S,),
        in_specs=[pl.BlockSpec((half, 128), lambda s: (s, 0))],
        out_specs=pl.BlockSpec((1, D_MODEL, B), lambda s: (s, 0, 0)),
        out_shape=jax.ShapeDtypeStruct((S, D_MODEL, B), jnp.float32),
        compiler_params=pltpu.CompilerParams(
            dimension_semantics=("parallel",),
        ),
    )(out_pairs)


def _sc_gather(x_flat, lut, S, B):
    n_idx = S * B
    n_chunks = n_idx // (NW * CHUNK)
    half = B // 2
    mesh = plsc.VectorSubcoreMesh(core_axis_name="c", subcore_axis_name="s")

    @functools.partial(
        pl.kernel,
        mesh=mesh,
        out_type=jax.ShapeDtypeStruct((n_idx // 2, 128), jnp.float32),
        compiler_params=pltpu.CompilerParams(use_tc_tiling_on_sc=False),
        scratch_types=[
            pltpu.VMEM((CHUNK,), jnp.int32),
            pltpu.VMEM((CHUNK, D_MODEL), jnp.float32),
            pltpu.SemaphoreType.DMA,
        ],
    )
    def k(lut_hbm, idx_hbm, out_hbm, idx_v, rows_v, sem):
        wid = lax.axis_index("s") * NC + lax.axis_index("c")

        @pl.loop(0, n_chunks)
        def _(g):
            kc = wid * n_chunks + g
            base = kc * CHUNK
            pltpu.sync_copy(idx_hbm.at[pl.ds(base, CHUNK)], idx_v)
            pltpu.async_copy(lut_hbm.at[idx_v], rows_v, sem).wait()

            s = kc // 4
            q = kc - s * 4
            h = q // 2
            pairbase = s * half + (q - h * 2) * CHUNK
            pltpu.sync_copy(
                rows_v,
                out_hbm.at[pl.ds(pairbase, CHUNK), pl.ds(h * D_MODEL, D_MODEL)],
            )

    return k(lut, x_flat)


def kernel(x, lut):
    B, S = x.shape
    x_t = jnp.swapaxes(x.astype(jnp.int32), 0, 1)
    x_flat = x_t.reshape(-1)
    out_pairs = _sc_gather(x_flat, lut, S, B)
    out_t = _tc_unpack_scale_out(out_pairs, S, B)
    return jnp.transpose(out_t, (2, 0, 1))


# speedup vs baseline: 1.0023x; 1.0023x over previous
import functools

import jax
import jax.numpy as jnp
from jax import lax
from jax.experimental import pallas as pl
from jax.experimental.pallas import tpu as pltpu
from jax.experimental.pallas import tpu_sc as plsc

D_MODEL = 64
SCALE = 8.0

NC = 2
NS = 16
NW = NC * NS

CHUNK = 1024


def _tc_prep_idx(x, S, B):
    def body(in_ref, out_ref):
        out_ref[...] = in_ref[...].T.reshape(S * B // 128, 128)

    return pl.pallas_call(
        body,
        in_specs=[pl.BlockSpec((B, S), lambda: (0, 0))],
        out_specs=pl.BlockSpec((S * B // 128, 128), lambda: (0, 0)),
        out_shape=jax.ShapeDtypeStruct((S * B // 128, 128), jnp.int32),
    )(x)


def _tc_unpack_scale_out(out_pairs, S, B):
    half = B // 2

    def body(in_ref, out_ref):
        x = in_ref[...]
        lo = x[:, :D_MODEL].T
        hi = x[:, D_MODEL:].T
        out_ref[...] = (jnp.concatenate([lo, hi], axis=1) * SCALE).reshape(
            1, D_MODEL, B
        )

    return pl.pallas_call(
        body,
        grid=(S,),
        in_specs=[pl.BlockSpec((half, 128), lambda s: (s, 0))],
        out_specs=pl.BlockSpec((1, D_MODEL, B), lambda s: (s, 0, 0)),
        out_shape=jax.ShapeDtypeStruct((S, D_MODEL, B), jnp.float32),
        compiler_params=pltpu.CompilerParams(
            dimension_semantics=("parallel",),
        ),
    )(out_pairs)


def _sc_gather(x_flat, lut, S, B):
    n_idx = S * B
    n_chunks = n_idx // (NW * CHUNK)
    half = B // 2
    mesh = plsc.VectorSubcoreMesh(core_axis_name="c", subcore_axis_name="s")

    @functools.partial(
        pl.kernel,
        mesh=mesh,
        out_type=jax.ShapeDtypeStruct((n_idx // 2, 128), jnp.float32),
        compiler_params=pltpu.CompilerParams(use_tc_tiling_on_sc=False),
        scratch_types=[
            pltpu.VMEM((CHUNK,), jnp.int32),
            pltpu.VMEM((CHUNK, D_MODEL), jnp.float32),
            pltpu.SemaphoreType.DMA,
        ],
    )
    def k(lut_hbm, idx_hbm, out_hbm, idx_v, rows_v, sem):
        wid = lax.axis_index("s") * NC + lax.axis_index("c")

        @pl.loop(0, n_chunks)
        def _(g):
            kc = wid * n_chunks + g
            base = kc * CHUNK
            pltpu.sync_copy(idx_hbm.at[pl.ds(base, CHUNK)], idx_v)
            pltpu.async_copy(lut_hbm.at[idx_v], rows_v, sem).wait()

            s = kc // 4
            q = kc - s * 4
            h = q // 2
            pairbase = s * half + (q - h * 2) * CHUNK
            pltpu.sync_copy(
                rows_v,
                out_hbm.at[pl.ds(pairbase, CHUNK), pl.ds(h * D_MODEL, D_MODEL)],
            )

    return k(lut, x_flat)


def kernel(x, lut):
    B, S = x.shape
    x_flat = _tc_prep_idx(x.astype(jnp.int32), S, B).reshape(-1)
    out_pairs = _sc_gather(x_flat, lut, S, B)
    out_t = _tc_unpack_scale_out(out_pairs, S, B)
    return jnp.transpose(out_t, (2, 0, 1))


# speedup vs baseline: 1.5439x; 1.5403x over previous
import functools

import jax
import jax.numpy as jnp
from jax import lax
from jax.experimental import pallas as pl
from jax.experimental.pallas import tpu as pltpu
from jax.experimental.pallas import tpu_sc as plsc

D_MODEL = 64
SCALE = 8.0

NC = 2
NS = 16
NW = NC * NS

LUT_BLK = 8192
LUT_HALF = LUT_BLK // 2
CHUNK = 1024


def _eye(n, dtype, scale=1.0):
    r = lax.broadcasted_iota(jnp.int32, (n, n), 0)
    c = lax.broadcasted_iota(jnp.int32, (n, n), 1)
    return jnp.where(r == c, jnp.asarray(scale, dtype), jnp.asarray(0, dtype))


def _tc_pack_lut(lut_t, vp):
    V = lut_t.shape[1]
    grid = (V + LUT_BLK - 1) // LUT_BLK

    def body(in_ref, out_ref):
        x = in_ref[...]
        t = lax.dot_general(
            x, _eye(D_MODEL, x.dtype),
            (((0,), (0,)), ((), ())),
            preferred_element_type=jnp.float32,
        )
        out_ref[...] = jnp.concatenate([t[:LUT_HALF], t[LUT_HALF:]], axis=1)

    return pl.pallas_call(
        body,
        grid=(grid,),
        in_specs=[pl.BlockSpec((D_MODEL, LUT_BLK), lambda i: (0, i))],
        out_specs=pl.BlockSpec((LUT_HALF, 128), lambda i: (i, 0)),
        out_shape=jax.ShapeDtypeStruct((vp // 2, 128), jnp.float32),
        compiler_params=pltpu.CompilerParams(
            dimension_semantics=("parallel",),
        ),
    )(lut_t)


def _tc_prep_idx(x, S, B):
    def body(in_ref, out_ref):
        out_ref[...] = in_ref[...].T.reshape(S * B // 128, 128)

    return pl.pallas_call(
        body,
        in_specs=[pl.BlockSpec((B, S), lambda: (0, 0))],
        out_specs=pl.BlockSpec((S * B // 128, 128), lambda: (0, 0)),
        out_shape=jax.ShapeDtypeStruct((S * B // 128, 128), jnp.int32),
    )(x)


def _tc_unpack_scale_out(out_pairs, S, B):
    half = B // 2

    def body(in_ref, out_ref):
        x = in_ref[...]
        y = lax.dot_general(
            _eye(128, x.dtype, SCALE), x,
            (((1,), (1,)), ((), ())),
            preferred_element_type=jnp.float32,
        )
        out_ref[...] = jnp.concatenate(
            [y[:D_MODEL], y[D_MODEL:]], axis=1
        ).reshape(1, D_MODEL, B)

    return pl.pallas_call(
        body,
        grid=(S,),
        in_specs=[pl.BlockSpec((half, 128), lambda s: (s, 0))],
        out_specs=pl.BlockSpec((1, D_MODEL, B), lambda s: (s, 0, 0)),
        out_shape=jax.ShapeDtypeStruct((S, D_MODEL, B), jnp.float32),
        compiler_params=pltpu.CompilerParams(
            dimension_semantics=("parallel",),
        ),
    )(out_pairs)


def _sc_gather(x_flat, lut_rows, S, B):
    n_idx = S * B
    n_chunks = n_idx // (NW * CHUNK)
    half = B // 2
    mesh = plsc.VectorSubcoreMesh(core_axis_name="c", subcore_axis_name="s")

    @functools.partial(
        pl.kernel,
        mesh=mesh,
        out_type=jax.ShapeDtypeStruct((n_idx // 2, 128), jnp.float32),
        compiler_params=pltpu.CompilerParams(use_tc_tiling_on_sc=False),
        scratch_types=[
            pltpu.VMEM((CHUNK,), jnp.int32),
            pltpu.VMEM((CHUNK, D_MODEL), jnp.float32),
            pltpu.SemaphoreType.DMA,
        ],
    )
    def k(lut_hbm, idx_hbm, out_hbm, idx_v, rows_v, sem):
        wid = lax.axis_index("s") * NC + lax.axis_index("c")

        @pl.loop(0, n_chunks)
        def _(g):
            kc = wid * n_chunks + g
            base = kc * CHUNK
            pltpu.sync_copy(idx_hbm.at[pl.ds(base, CHUNK)], idx_v)

            @pl.loop(0, CHUNK, step=16)
            def _(o):
                v = idx_v.at[pl.ds(o, 16)][...]
                u = jnp.bitwise_and(v, LUT_BLK - 1)
                j = (
                    (v - u)
                    + jnp.left_shift(jnp.bitwise_and(u, LUT_HALF - 1), 1)
                    + jnp.right_shift(u, 12)
                )
                idx_v.at[pl.ds(o, 16)][...] = j

            pltpu.async_copy(lut_hbm.at[idx_v], rows_v, sem).wait()

            s = kc // 4
            q = kc - s * 4
            h = q // 2
            pairbase = s * half + (q - h * 2) * CHUNK
            pltpu.sync_copy(
                rows_v,
                out_hbm.at[pl.ds(pairbase, CHUNK), pl.ds(h * D_MODEL, D_MODEL)],
            )

    return k(lut_rows, x_flat)


def kernel(x, lut):
    B, S = x.shape
    V = lut.shape[0]
    n_blocks = (V + LUT_BLK - 1) // LUT_BLK
    vp = n_blocks * LUT_BLK
    x_flat = _tc_prep_idx(x.astype(jnp.int32), S, B).reshape(-1)
    lut_t = jnp.swapaxes(lut, 0, 1)
    lut_pairs = _tc_pack_lut(lut_t, vp)
    lut_rows = lut_pairs.reshape(vp, D_MODEL)
    out_pairs = _sc_gather(x_flat, lut_rows, S, B)
    out_t = _tc_unpack_scale_out(out_pairs, S, B)
    return jnp.transpose(out_t, (2, 0, 1))


# speedup vs baseline: 1.8257x; 1.1825x over previous
import functools

import jax
import jax.numpy as jnp
from jax import lax
from jax.experimental import pallas as pl
from jax.experimental.pallas import tpu as pltpu
from jax.experimental.pallas import tpu_sc as plsc

D_MODEL = 64
SCALE = 8.0

NC = 2
NS = 16
NW = NC * NS

LUT_BLK = 16384
LUT_HALF = LUT_BLK // 2
LUT_SHIFT = LUT_HALF.bit_length() - 1
CHUNK = 1024
N_SLICE = 5


def _eye(n, dtype, scale=1.0):
    r = lax.broadcasted_iota(jnp.int32, (n, n), 0)
    c = lax.broadcasted_iota(jnp.int32, (n, n), 1)
    return jnp.where(r == c, jnp.asarray(scale, dtype), jnp.asarray(0, dtype))


def _tc_pack_lut(lut_t, vp):
    grid = vp // LUT_BLK

    def body(in_ref, out_ref):
        x = in_ref[...]
        t = lax.dot_general(
            x, _eye(D_MODEL, x.dtype),
            (((0,), (0,)), ((), ())),
            preferred_element_type=jnp.float32,
        )
        out_ref[...] = jnp.concatenate([t[:LUT_HALF], t[LUT_HALF:]], axis=1)

    return pl.pallas_call(
        body,
        grid=(grid,),
        in_specs=[pl.BlockSpec((D_MODEL, LUT_BLK), lambda i: (0, i))],
        out_specs=pl.BlockSpec((LUT_HALF, 128), lambda i: (i, 0)),
        out_shape=jax.ShapeDtypeStruct((vp // 2, 128), jnp.float32),
        compiler_params=pltpu.CompilerParams(
            dimension_semantics=("parallel",),
        ),
    )(lut_t)


def _tc_prep_idx(x, S, B):
    def body(in_ref, out_ref):
        out_ref[...] = in_ref[...].T.reshape(S * B // 128, 128)

    return pl.pallas_call(
        body,
        in_specs=[pl.BlockSpec((B, S), lambda: (0, 0))],
        out_specs=pl.BlockSpec((S * B // 128, 128), lambda: (0, 0)),
        out_shape=jax.ShapeDtypeStruct((S * B // 128, 128), jnp.int32),
    )(x)


def _tc_unpack_slice(acc, pairs_i, i, ssl, S, B):
    half = B // 2

    def body(*refs):
        in_ref, out_ref = refs[-2], refs[-1]
        x = in_ref[...]
        y = lax.dot_general(
            _eye(128, x.dtype, SCALE), x,
            (((1,), (1,)), ((), ())),
            preferred_element_type=jnp.float32,
        )
        out_ref[...] = jnp.concatenate(
            [y[:D_MODEL], y[D_MODEL:]], axis=1
        ).reshape(1, D_MODEL, B)

    in_specs = [pl.BlockSpec((half, 128), lambda s: (s, 0))]
    operands = [pairs_i]
    kwargs = {}
    if acc is not None:
        in_specs = [pl.BlockSpec(memory_space=pl.ANY)] + in_specs
        operands = [acc] + operands
        kwargs["input_output_aliases"] = {0: 0}

    return pl.pallas_call(
        body,
        grid=(ssl,),
        in_specs=in_specs,
        out_specs=pl.BlockSpec(
            (1, D_MODEL, B), lambda s, i=i: (i * ssl + s, 0, 0)
        ),
        out_shape=jax.ShapeDtypeStruct((S, D_MODEL, B), jnp.float32),
        compiler_params=pltpu.CompilerParams(
            dimension_semantics=("arbitrary",),
        ),
        **kwargs,
    )(*operands)


def _sc_gather_slice(x_flat, lut_rows, i, ssl, B):
    n_idx = ssl * B
    n_chunks = n_idx // (NW * CHUNK)
    half = B // 2
    idx0 = i * n_idx // CHUNK
    mesh = plsc.VectorSubcoreMesh(core_axis_name="c", subcore_axis_name="s")

    @functools.partial(
        pl.kernel,
        mesh=mesh,
        out_type=jax.ShapeDtypeStruct((n_idx // 2, 128), jnp.float32),
        compiler_params=pltpu.CompilerParams(use_tc_tiling_on_sc=False),
        scratch_types=[
            pltpu.VMEM((CHUNK,), jnp.int32),
            pltpu.VMEM((CHUNK, D_MODEL), jnp.float32),
            pltpu.SemaphoreType.DMA,
        ],
    )
    def k(lut_hbm, idx_hbm, out_hbm, idx_v, rows_v, sem):
        wid = lax.axis_index("s") * NC + lax.axis_index("c")

        @pl.loop(0, n_chunks)
        def _(g):
            kc = wid * n_chunks + g
            pltpu.sync_copy(idx_hbm.at[pl.ds((idx0 + kc) * CHUNK, CHUNK)], idx_v)

            @pl.loop(0, CHUNK, step=16)
            def _(o):
                v = idx_v.at[pl.ds(o, 16)][...]
                u = jnp.bitwise_and(v, LUT_BLK - 1)
                j = (
                    (v - u)
                    + jnp.left_shift(jnp.bitwise_and(u, LUT_HALF - 1), 1)
                    + jnp.right_shift(u, LUT_SHIFT)
                )
                idx_v.at[pl.ds(o, 16)][...] = j

            pltpu.async_copy(lut_hbm.at[idx_v], rows_v, sem).wait()

            s = kc // 4
            q = kc - s * 4
            h = q // 2
            pairbase = s * half + (q - h * 2) * CHUNK
            pltpu.sync_copy(
                rows_v,
                out_hbm.at[pl.ds(pairbase, CHUNK), pl.ds(h * D_MODEL, D_MODEL)],
            )

    return k(lut_rows, x_flat)


def kernel(x, lut):
    B, S = x.shape
    V = lut.shape[0]
    n_blocks = (V + LUT_BLK - 1) // LUT_BLK
    vp = n_blocks * LUT_BLK
    ssl = S // N_SLICE
    x_flat = _tc_prep_idx(x.astype(jnp.int32), S, B).reshape(-1)
    lut_t = jnp.swapaxes(lut, 0, 1)
    lut_pairs = _tc_pack_lut(lut_t, vp)
    lut_rows = lut_pairs.reshape(vp, D_MODEL)
    pairs = [
        _sc_gather_slice(x_flat, lut_rows, i, ssl, B) for i in range(N_SLICE)
    ]
    acc = None
    for i in range(N_SLICE):
        acc = _tc_unpack_slice(acc, pairs[i], i, ssl, S, B)
    return jnp.transpose(acc, (2, 0, 1))


# speedup vs baseline: 1.8398x; 1.0078x over previous
import functools

import jax
import jax.numpy as jnp
from jax import lax
from jax.experimental import pallas as pl
from jax.experimental.pallas import tpu as pltpu
from jax.experimental.pallas import tpu_sc as plsc

D_MODEL = 64
SCALE = 8.0

NC = 2
NS = 16
NW = NC * NS

LUT_BLK = 32768
LUT_HALF = LUT_BLK // 2
LUT_SHIFT = LUT_HALF.bit_length() - 1
CHUNK = 1024
SLICES = (16, 56, 56, 48, 24)


def _eye(n, dtype, scale=1.0):
    r = lax.broadcasted_iota(jnp.int32, (n, n), 0)
    c = lax.broadcasted_iota(jnp.int32, (n, n), 1)
    return jnp.where(r == c, jnp.asarray(scale, dtype), jnp.asarray(0, dtype))


def _tc_pack_lut(lut_t, vp):
    grid = vp // LUT_BLK

    def body(in_ref, out_ref):
        x = in_ref[...]
        t = lax.dot_general(
            x, _eye(D_MODEL, x.dtype),
            (((0,), (0,)), ((), ())),
            preferred_element_type=jnp.float32,
        )
        out_ref[...] = jnp.concatenate([t[:LUT_HALF], t[LUT_HALF:]], axis=1)

    return pl.pallas_call(
        body,
        grid=(grid,),
        in_specs=[pl.BlockSpec((D_MODEL, LUT_BLK), lambda i: (0, i))],
        out_specs=pl.BlockSpec((LUT_HALF, 128), lambda i: (i, 0)),
        out_shape=jax.ShapeDtypeStruct((vp // 2, 128), jnp.float32),
        compiler_params=pltpu.CompilerParams(
            dimension_semantics=("parallel",),
        ),
    )(lut_t)


def _tc_prep_idx(x, S, B):
    def body(in_ref, out_ref):
        v = in_ref[...].T.reshape(S * B // 128, 128)
        u = jnp.bitwise_and(v, LUT_BLK - 1)
        out_ref[...] = (
            (v - u)
            + jnp.left_shift(jnp.bitwise_and(u, LUT_HALF - 1), 1)
            + jnp.right_shift(u, LUT_SHIFT)
        )

    return pl.pallas_call(
        body,
        in_specs=[pl.BlockSpec((B, S), lambda: (0, 0))],
        out_specs=pl.BlockSpec((S * B // 128, 128), lambda: (0, 0)),
        out_shape=jax.ShapeDtypeStruct((S * B // 128, 128), jnp.int32),
    )(x)


def _tc_unpack_slice(acc, pairs_i, off, ssl, S, B):
    half = B // 2

    def body(*refs):
        in_ref, out_ref = refs[-2], refs[-1]
        x = in_ref[...]
        y = lax.dot_general(
            _eye(128, x.dtype, SCALE), x,
            (((1,), (1,)), ((), ())),
            preferred_element_type=jnp.float32,
        )
        out_ref[...] = jnp.concatenate(
            [y[:D_MODEL], y[D_MODEL:]], axis=1
        ).reshape(1, D_MODEL, B)

    in_specs = [pl.BlockSpec((half, 128), lambda s: (s, 0))]
    operands = [pairs_i]
    kwargs = {}
    if acc is not None:
        in_specs = [pl.BlockSpec(memory_space=pl.ANY)] + in_specs
        operands = [acc] + operands
        kwargs["input_output_aliases"] = {0: 0}

    return pl.pallas_call(
        body,
        grid=(ssl,),
        in_specs=in_specs,
        out_specs=pl.BlockSpec(
            (1, D_MODEL, B), lambda s, off=off: (off + s, 0, 0)
        ),
        out_shape=jax.ShapeDtypeStruct((S, D_MODEL, B), jnp.float32),
        compiler_params=pltpu.CompilerParams(
            dimension_semantics=("arbitrary",),
        ),
        **kwargs,
    )(*operands)


def _sc_gather_slice(x_flat, lut_rows, off, ssl, B):
    n_idx = ssl * B
    n_chunks = n_idx // (NW * CHUNK)
    half = B // 2
    idx0 = off * B // CHUNK
    mesh = plsc.VectorSubcoreMesh(core_axis_name="c", subcore_axis_name="s")

    @functools.partial(
        pl.kernel,
        mesh=mesh,
        out_type=jax.ShapeDtypeStruct((n_idx // 2, 128), jnp.float32),
        compiler_params=pltpu.CompilerParams(use_tc_tiling_on_sc=False),
        scratch_types=[
            pltpu.VMEM((CHUNK,), jnp.int32),
            pltpu.VMEM((CHUNK, D_MODEL), jnp.float32),
            pltpu.SemaphoreType.DMA,
        ],
    )
    def k(lut_hbm, idx_hbm, out_hbm, idx_v, rows_v, sem):
        wid = lax.axis_index("s") * NC + lax.axis_index("c")

        @pl.loop(0, n_chunks)
        def _(g):
            kc = wid * n_chunks + g
            pltpu.sync_copy(idx_hbm.at[pl.ds((idx0 + kc) * CHUNK, CHUNK)], idx_v)
            pltpu.async_copy(lut_hbm.at[idx_v], rows_v, sem).wait()

            s = kc // 4
            q = kc - s * 4
            h = q // 2
            pairbase = s * half + (q - h * 2) * CHUNK
            pltpu.sync_copy(
                rows_v,
                out_hbm.at[pl.ds(pairbase, CHUNK), pl.ds(h * D_MODEL, D_MODEL)],
            )

    return k(lut_rows, x_flat)


def kernel(x, lut):
    B, S = x.shape
    V = lut.shape[0]
    n_blocks = (V + LUT_BLK - 1) // LUT_BLK
    vp = n_blocks * LUT_BLK
    x_flat = _tc_prep_idx(x.astype(jnp.int32), S, B).reshape(-1)
    lut_t = jnp.swapaxes(lut, 0, 1)
    lut_pairs = _tc_pack_lut(lut_t, vp)
    lut_rows = lut_pairs.reshape(vp, D_MODEL)
    offs = [sum(SLICES[:i]) for i in range(len(SLICES))]
    pairs = [
        _sc_gather_slice(x_flat, lut_rows, off, ssl, B)
        for off, ssl in zip(offs, SLICES)
    ]
    acc = None
    for p, off, ssl in zip(pairs, offs, SLICES):
        acc = _tc_unpack_slice(acc, p, off, ssl, S, B)
    return jnp.transpose(acc, (2, 0, 1))


# speedup vs baseline: 1.8637x; 1.0129x over previous
import functools

import jax
import jax.numpy as jnp
from jax import lax
from jax.experimental import pallas as pl
from jax.experimental.pallas import tpu as pltpu
from jax.experimental.pallas import tpu_sc as plsc

D_MODEL = 64
SCALE = 8.0

NC = 2
NS = 16
NW = NC * NS

LUT_BLK = 32768
LUT_HALF = LUT_BLK // 2
LUT_SHIFT = LUT_HALF.bit_length() - 1
CHUNK = 1024
SLICES = (16, 56, 56, 56, 16)


def _eye(n, dtype, scale=1.0):
    r = lax.broadcasted_iota(jnp.int32, (n, n), 0)
    c = lax.broadcasted_iota(jnp.int32, (n, n), 1)
    return jnp.where(r == c, jnp.asarray(scale, dtype), jnp.asarray(0, dtype))


def _tc_pack_lut(lut_t, vp):
    grid = vp // LUT_BLK

    def body(in_ref, out_ref):
        x = in_ref[...]
        t = lax.dot_general(
            x, _eye(D_MODEL, x.dtype),
            (((0,), (0,)), ((), ())),
            preferred_element_type=jnp.float32,
        )
        out_ref[...] = jnp.concatenate([t[:LUT_HALF], t[LUT_HALF:]], axis=1)

    return pl.pallas_call(
        body,
        grid=(grid,),
        in_specs=[pl.BlockSpec((D_MODEL, LUT_BLK), lambda i: (0, i))],
        out_specs=pl.BlockSpec((LUT_HALF, 128), lambda i: (i, 0)),
        out_shape=jax.ShapeDtypeStruct((vp // 2, 128), jnp.float32),
        compiler_params=pltpu.CompilerParams(
            dimension_semantics=("parallel",),
        ),
    )(lut_t)


def _tc_prep_idx(x_t, S, B):
    def body(in_ref, out_ref):
        v = in_ref[...].reshape(S * B // 128, 128)
        u = jnp.bitwise_and(v, LUT_BLK - 1)
        out_ref[...] = (
            (v - u)
            + jnp.left_shift(jnp.bitwise_and(u, LUT_HALF - 1), 1)
            + jnp.right_shift(u, LUT_SHIFT)
        )

    return pl.pallas_call(
        body,
        in_specs=[pl.BlockSpec((S, B), lambda: (0, 0))],
        out_specs=pl.BlockSpec((S * B // 128, 128), lambda: (0, 0)),
        out_shape=jax.ShapeDtypeStruct((S * B // 128, 128), jnp.int32),
    )(x_t)


def _tc_unpack_slice(acc, pairs_i, off, ssl, S, B):
    half = B // 2

    def body(*refs):
        in_ref, out_ref = refs[-2], refs[-1]
        x = in_ref[...]
        y = lax.dot_general(
            _eye(128, x.dtype, SCALE), x,
            (((1,), (1,)), ((), ())),
            preferred_element_type=jnp.float32,
        )
        out_ref[...] = jnp.concatenate(
            [y[:D_MODEL], y[D_MODEL:]], axis=1
        ).reshape(1, D_MODEL, B)

    in_specs = [pl.BlockSpec((half, 128), lambda s: (s, 0))]
    operands = [pairs_i]
    kwargs = {}
    if acc is not None:
        in_specs = [pl.BlockSpec(memory_space=pl.ANY)] + in_specs
        operands = [acc] + operands
        kwargs["input_output_aliases"] = {0: 0}

    return pl.pallas_call(
        body,
        grid=(ssl,),
        in_specs=in_specs,
        out_specs=pl.BlockSpec(
            (1, D_MODEL, B), lambda s, off=off: (off + s, 0, 0)
        ),
        out_shape=jax.ShapeDtypeStruct((S, D_MODEL, B), jnp.float32),
        compiler_params=pltpu.CompilerParams(
            dimension_semantics=("arbitrary",),
        ),
        **kwargs,
    )(*operands)


def _sc_gather_slice(x_flat, lut_rows, off, ssl, B):
    n_idx = ssl * B
    n_chunks = n_idx // (NW * CHUNK)
    half = B // 2
    idx0 = off * B // CHUNK
    mesh = plsc.VectorSubcoreMesh(core_axis_name="c", subcore_axis_name="s")

    @functools.partial(
        pl.kernel,
        mesh=mesh,
        out_type=jax.ShapeDtypeStruct((n_idx // 2, 128), jnp.float32),
        compiler_params=pltpu.CompilerParams(use_tc_tiling_on_sc=False),
        scratch_types=[
            pltpu.VMEM((CHUNK,), jnp.int32),
            pltpu.VMEM((CHUNK, D_MODEL), jnp.float32),
            pltpu.SemaphoreType.DMA,
        ],
    )
    def k(lut_hbm, idx_hbm, out_hbm, idx_v, rows_v, sem):
        wid = lax.axis_index("s") * NC + lax.axis_index("c")

        @pl.loop(0, n_chunks)
        def _(g):
            kc = wid * n_chunks + g
            pltpu.sync_copy(idx_hbm.at[pl.ds((idx0 + kc) * CHUNK, CHUNK)], idx_v)
            pltpu.async_copy(lut_hbm.at[idx_v], rows_v, sem).wait()

            s = kc // 4
            q = kc - s * 4
            h = q // 2
            pairbase = s * half + (q - h * 2) * CHUNK
            pltpu.sync_copy(
                rows_v,
                out_hbm.at[pl.ds(pairbase, CHUNK), pl.ds(h * D_MODEL, D_MODEL)],
            )

    return k(lut_rows, x_flat)


def kernel(x, lut):
    B, S = x.shape
    V = lut.shape[0]
    n_blocks = (V + LUT_BLK - 1) // LUT_BLK
    vp = n_blocks * LUT_BLK
    x_t = jnp.swapaxes(x.astype(jnp.int32), 0, 1)
    x_flat = _tc_prep_idx(x_t, S, B).reshape(-1)
    lut_t = jnp.swapaxes(lut, 0, 1)
    lut_pairs = _tc_pack_lut(lut_t, vp)
    lut_rows = lut_pairs.reshape(vp, D_MODEL)
    offs = [sum(SLICES[:i]) for i in range(len(SLICES))]
    pairs = [
        _sc_gather_slice(x_flat, lut_rows, off, ssl, B)
        for off, ssl in zip(offs, SLICES)
    ]
    acc = None
    for p, off, ssl in zip(pairs, offs, SLICES):
        acc = _tc_unpack_slice(acc, p, off, ssl, S, B)
    return jnp.transpose(acc, (2, 0, 1))
